# Initial kernel scaffold; baseline (speedup 1.0000x reference)
import jax
import jax.numpy as jnp
from jax.experimental import pallas as pl


def kernel(x, table):
    raise NotImplementedError("write your pallas kernel here")


# speedup vs baseline: 5.6343x; 5.6343x over previous
import functools

import jax
import jax.numpy as jnp
from jax import lax
from jax.experimental import pallas as pl
from jax.experimental.pallas import tpu as pltpu
from jax.experimental.pallas import tpu_sc as plsc

ROWS, COLS = 16384, 200
N = ROWS * COLS
NW = 32
PER_W = N // NW
CHUNK = 12800
NCH = PER_W // CHUNK
GROUPS = CHUNK // 16

_mesh = plsc.VectorSubcoreMesh(core_axis_name="c", subcore_axis_name="s")


@functools.partial(
    pl.kernel,
    out_type=jax.ShapeDtypeStruct((N * 3,), jnp.float32),
    mesh=_mesh,
    scratch_types=[
        pltpu.VMEM((32,), jnp.float32),
        pltpu.VMEM((CHUNK,), jnp.int32),
        pltpu.VMEM((CHUNK * 3,), jnp.float32),
    ],
    compiler_params=pltpu.CompilerParams(needs_layout_passes=False),
)
def _emb_lookup(x_hbm, t_hbm, out_hbm, t_v, idx_v, out_v):
    wid = lax.axis_index("s") * 2 + lax.axis_index("c")
    base = wid * PER_W

    pltpu.sync_copy(t_hbm, t_v)

    lane3 = lax.iota(jnp.int32, 16) * 3

    def chunk_body(g, carry):
        pltpu.sync_copy(x_hbm.at[pl.ds(base + g * CHUNK, CHUNK)], idx_v)

        def group_body(i, c):
            iv = idx_v[pl.ds(i * 16, 16)]
            b = iv * 3
            t0 = plsc.load_gather(t_v, [b])
            t1 = plsc.load_gather(t_v, [b + 1])
            t2 = plsc.load_gather(t_v, [b + 2])
            pos = lane3 + i * 48
            plsc.store_scatter(out_v, [pos], t0)
            plsc.store_scatter(out_v, [pos + 1], t1)
            plsc.store_scatter(out_v, [pos + 2], t2)
            return c

        lax.fori_loop(0, GROUPS, group_body, 0, unroll=4)
        pltpu.sync_copy(
            out_v, out_hbm.at[pl.ds((base + g * CHUNK) * 3, CHUNK * 3)]
        )
        return carry

    lax.fori_loop(0, NCH, chunk_body, 0)


def kernel(x, table):
    xf = x.reshape(-1)
    tf = jnp.pad(table.reshape(-1), (0, 2))
    out = _emb_lookup(xf, tf)
    return out.reshape(ROWS, COLS, 3)


# speedup vs baseline: 5.7938x; 1.0283x over previous
import functools

import jax
import jax.numpy as jnp
from jax import lax
from jax.experimental import pallas as pl
from jax.experimental.pallas import tpu as pltpu
from jax.experimental.pallas import tpu_sc as plsc

ROWS, COLS = 16384, 200
N = ROWS * COLS
NW = 32
PER_W = N // NW
CHUNK = 12800
NCH = PER_W // CHUNK
GROUPS = CHUNK // 16

_mesh = plsc.VectorSubcoreMesh(core_axis_name="c", subcore_axis_name="s")


@functools.partial(
    pl.kernel,
    out_type=jax.ShapeDtypeStruct((N * 3,), jnp.float32),
    mesh=_mesh,
    scratch_types=[
        pltpu.VMEM((32,), jnp.float32),
        pltpu.VMEM((CHUNK,), jnp.int32),
        pltpu.VMEM((CHUNK * 3,), jnp.float32),
    ],
    compiler_params=pltpu.CompilerParams(needs_layout_passes=False),
)
def _emb_lookup(x_hbm, t_hbm, out_hbm, t_v, idx_v, out_v):
    wid = lax.axis_index("s") * 2 + lax.axis_index("c")
    base = wid * PER_W

    pltpu.sync_copy(t_hbm, t_v)

    lane3 = lax.iota(jnp.int32, 16) * 3

    def chunk_body(g, carry):
        pltpu.sync_copy(x_hbm.at[pl.ds(base + g * CHUNK, CHUNK)], idx_v)

        @plsc.parallel_loop(0, GROUPS, 1, unroll=8)
        def group_body(i):
            iv = idx_v[pl.ds(i * 16, 16)]
            b = iv * 3
            t0 = plsc.load_gather(t_v, [b])
            t1 = plsc.load_gather(t_v, [b + 1])
            t2 = plsc.load_gather(t_v, [b + 2])
            pos = lane3 + i * 48
            plsc.store_scatter(out_v, [pos], t0)
            plsc.store_scatter(out_v, [pos + 1], t1)
            plsc.store_scatter(out_v, [pos + 2], t2)
        pltpu.sync_copy(
            out_v, out_hbm.at[pl.ds((base + g * CHUNK) * 3, CHUNK * 3)]
        )
        return carry

    lax.fori_loop(0, NCH, chunk_body, 0)


def kernel(x, table):
    xf = x.reshape(-1)
    tf = jnp.pad(table.reshape(-1), (0, 2))
    out = _emb_lookup(xf, tf)
    return out.reshape(ROWS, COLS, 3)


# speedup vs baseline: 166.7452x; 28.7802x over previous
import functools

import jax
import jax.numpy as jnp
from jax import lax
from jax.experimental import pallas as pl
from jax.experimental.pallas import tpu as pltpu
from jax.experimental.pallas import tpu_sc as plsc

ROWS, COLS = 16384, 200
NW = 32
W = ROWS // NW
TR = COLS // 8
GPR = W // 16

_mesh = plsc.VectorSubcoreMesh(core_axis_name="c", subcore_axis_name="s")


@functools.partial(
    pl.kernel,
    out_type=jax.ShapeDtypeStruct((3, COLS, ROWS), jnp.float32),
    mesh=_mesh,
    scratch_types=[
        pltpu.VMEM((48,), jnp.float32),
        pltpu.VMEM((8, W), jnp.int32),
        pltpu.VMEM((3, 8, W), jnp.float32),
    ],
    compiler_params=pltpu.CompilerParams(
        needs_layout_passes=False, use_tc_tiling_on_sc=True
    ),
)
def _emb_lookup(xt_hbm, t_hbm, out_hbm, t_v, idx_v, out_v):
    wid = lax.axis_index("s") * 2 + lax.axis_index("c")
    c0 = wid * W

    pltpu.sync_copy(t_hbm, t_v)

    def row_body(r, carry):
        pltpu.sync_copy(xt_hbm.at[pl.ds(r * 8, 8), pl.ds(c0, W)], idx_v)

        @plsc.parallel_loop(0, GPR, 1, unroll=2)
        def group_body(g):
            col = g * 16
            for s in range(8):
                iv = idx_v[s, pl.ds(col, 16)]
                for k in range(3):
                    tk = plsc.load_gather(t_v.at[pl.ds(k * 16, 16)], [iv])
                    out_v[k, s, pl.ds(col, 16)] = tk

        for k in range(3):
            pltpu.sync_copy(
                out_v.at[k], out_hbm.at[k, pl.ds(r * 8, 8), pl.ds(c0, W)]
            )
        return carry

    lax.fori_loop(0, TR, row_body, 0)


def kernel(x, table):
    tp = jnp.pad(table.T, ((0, 0), (0, 6))).reshape(-1)
    out = _emb_lookup(x.T, tp)
    return out.transpose(2, 1, 0)
